# baseline (speedup 1.0000x reference)
import functools

import jax
import jax.numpy as jnp
from jax import lax
from jax.experimental import pallas as pl
from jax.experimental.pallas import tpu as pltpu
from jax.experimental.pallas import tpu_sc as plsc

N = 160000
D = 256
NSEG = 64

N_SC = 76800
N_TC = N - N_SC

NC = 2
NS = 16
NW = NC * NS
R = N_SC // NW
CHUNK = 200
NCHUNK = R // CHUNK
NPAIR = NCHUNK // 2
LANES = 16
DJ = D // LANES
G = 8

TBLK = 1600
NBLK = N_TC // TBLK

NEG_INF = float("-inf")

assert R * NW == N_SC and NCHUNK * CHUNK == R and NBLK * TBLK == N_TC


def _tile_body(feat_hbm, ids_hbm, out_hbm,
               ids_v, buf0, buf1, acc_v, cur_v, sem0, sem1):
    c = lax.axis_index("c")
    s = lax.axis_index("s")
    w = s * NC + c
    base = w * R

    pltpu.sync_copy(ids_hbm.at[pl.ds(base, R)], ids_v.at[pl.ds(0, R)])

    def init_body(i, carry):
        acc_v[pl.ds(i * LANES, LANES)] = jnp.full((LANES,), NEG_INF, jnp.float32)
        return carry
    lax.fori_loop(0, (NSEG * D) // LANES, init_body, 0)
    for j in range(DJ):
        cur_v[pl.ds(j * LANES, LANES)] = jnp.full((LANES,), NEG_INF, jnp.float32)

    def flush(cs):
        for j in range(DJ):
            sa = pl.ds(cs * D + j * LANES, LANES)
            sc = pl.ds(j * LANES, LANES)
            acc_v[sa] = jnp.maximum(acc_v[sa], cur_v[sc])
            cur_v[sc] = jnp.full((LANES,), NEG_INF, jnp.float32)

    def process(buf, goff, cs):
        cfirst = ids_v[pl.ds(goff, LANES)][0]
        clast = ids_v[pl.ds(goff + CHUNK - 1, LANES)][0]

        def fast_chunk(cs):
            @pl.when(cfirst != cs)
            def _():
                flush(cs)

            def grp(k, accs):
                out = []
                for j in range(DJ):
                    a = accs[j]
                    for r in range(G):
                        a = jnp.maximum(
                            a, buf[k * G + r, pl.ds(j * LANES, LANES)])
                    out.append(a)
                return tuple(out)

            accs = tuple(cur_v[pl.ds(j * LANES, LANES)] for j in range(DJ))
            accs = lax.fori_loop(0, CHUNK // G, grp, accs)
            for j in range(DJ):
                cur_v[pl.ds(j * LANES, LANES)] = accs[j]
            return cfirst

        def slow_chunk(cs):
            def group_body(k, cs):
                ids16 = ids_v[pl.ds(goff + k * G, LANES)]
                first = ids16[0]
                last = ids16[G - 1]

                def fast(cs):
                    @pl.when(first != cs)
                    def _():
                        flush(cs)
                    for j in range(DJ):
                        sl = pl.ds(j * LANES, LANES)
                        a = cur_v[sl]
                        for r in range(G):
                            a = jnp.maximum(
                                a, buf[k * G + r, pl.ds(j * LANES, LANES)])
                        cur_v[sl] = a
                    return first

                def slow(cs):
                    for r in range(G):
                        soff = ids16[r] * D
                        for j in range(DJ):
                            sl = pl.ds(soff + j * LANES, LANES)
                            acc_v[sl] = jnp.maximum(
                                acc_v[sl],
                                buf[k * G + r, pl.ds(j * LANES, LANES)])
                    return cs

                return lax.cond(first == last, fast, slow, cs)
            return lax.fori_loop(0, CHUNK // G, group_body, cs)

        return lax.cond(cfirst == clast, fast_chunk, slow_chunk, cs)

    pltpu.async_copy(feat_hbm.at[pl.ds(base, CHUNK)], buf0, sem0)

    cs = ids_v[pl.ds(0, LANES)][0]

    def chunk_pair(g, cs):
        pltpu.async_copy(
            feat_hbm.at[pl.ds(base + (2 * g + 1) * CHUNK, CHUNK)], buf1, sem1)
        pltpu.make_async_copy(
            feat_hbm.at[pl.ds(base, CHUNK)], buf0, sem0).wait()
        cs = process(buf0, 2 * g * CHUNK, cs)

        @pl.when(2 * g + 2 < NCHUNK)
        def _():
            pltpu.async_copy(
                feat_hbm.at[pl.ds(base + (2 * g + 2) * CHUNK, CHUNK)],
                buf0, sem0)

        pltpu.make_async_copy(
            feat_hbm.at[pl.ds(base, CHUNK)], buf1, sem1).wait()
        cs = process(buf1, (2 * g + 1) * CHUNK, cs)
        return cs

    cs = lax.fori_loop(0, NPAIR, chunk_pair, cs)

    if NCHUNK % 2 == 1:
        pltpu.make_async_copy(feat_hbm.at[pl.ds(base, CHUNK)], buf0, sem0).wait()
        cs = process(buf0, (NCHUNK - 1) * CHUNK, cs)
    flush(cs)

    pltpu.sync_copy(acc_v, out_hbm.at[w])


@functools.partial(
    pl.kernel,
    out_type=jax.ShapeDtypeStruct((NW, NSEG * D), jnp.float32),
    mesh=plsc.VectorSubcoreMesh(core_axis_name="c", subcore_axis_name="s"),
    scratch_types=[
        pltpu.VMEM((R + LANES,), jnp.int32),
        pltpu.VMEM((CHUNK, D), jnp.float32),
        pltpu.VMEM((CHUNK, D), jnp.float32),
        pltpu.VMEM((NSEG * D,), jnp.float32),
        pltpu.VMEM((D,), jnp.float32),
        pltpu.SemaphoreType.DMA,
        pltpu.SemaphoreType.DMA,
    ],
)
def _segmax_sc(feat_hbm, ids_hbm, out_hbm,
               ids_v, buf0, buf1, acc_v, cur_v, sem0, sem1):
    _tile_body(feat_hbm, ids_hbm, out_hbm,
               ids_v, buf0, buf1, acc_v, cur_v, sem0, sem1)


def _tc_body(ids_ref, feat_ref, out_ref):
    i = pl.program_id(0)

    @pl.when(i == 0)
    def _():
        out_ref[...] = jnp.full((NSEG, D), NEG_INF, jnp.float32)

    ids_col = ids_ref[0]
    lo = jnp.min(ids_col)
    hi = jnp.max(ids_col)

    rows = lax.broadcasted_iota(jnp.int32, (NSEG, D), 0)

    def fold(seg, smax):
        cur = out_ref[...]
        out_ref[...] = jnp.where(rows == seg, jnp.maximum(cur, smax), cur)

    def uniform():
        fold(lo, jnp.max(feat_ref[...], axis=0, keepdims=True))

    def general():
        x = feat_ref[...]

        def seg_body(s, carry):
            xm = jnp.where(ids_col == s, x, NEG_INF)
            fold(s, jnp.max(xm, axis=0, keepdims=True))
            return carry
        lax.fori_loop(lo, hi + 1, seg_body, 0)

    lax.cond(lo == hi, uniform, general)


def _segmax_tc(feat_tc, ids_col3):
    return pl.pallas_call(
        _tc_body,
        grid=(NBLK,),
        in_specs=[
            pl.BlockSpec((1, TBLK, 1), lambda i: (i, 0, 0)),
            pl.BlockSpec((TBLK, D), lambda i: (i, 0)),
        ],
        out_specs=pl.BlockSpec((NSEG, D), lambda i: (0, 0)),
        out_shape=jax.ShapeDtypeStruct((NSEG, D), jnp.float32),
    )(ids_col3, feat_tc)


def kernel(features, segment_ids):
    ids32 = segment_ids.astype(jnp.int32)
    parts = _segmax_sc(features, ids32)
    sc_out = jnp.max(parts.reshape(NW, NSEG, D), axis=0)
    ids_col3 = ids32[N_SC:].reshape(NBLK, TBLK, 1)
    tc_out = _segmax_tc(features[N_SC:], ids_col3)
    return jnp.maximum(sc_out, tc_out)


# speedup vs baseline: 2.2206x; 2.2206x over previous
import functools

import jax
import jax.numpy as jnp
from jax import lax
from jax.experimental import pallas as pl
from jax.experimental.pallas import tpu as pltpu
from jax.experimental.pallas import tpu_sc as plsc

N = 160000
D = 256
NSEG = 64

N_SC = 76800
N_TC = N - N_SC

NC = 2
NS = 16
NW = NC * NS
R = N_SC // NW
CHUNK = 200
NCHUNK = R // CHUNK
NPAIR = NCHUNK // 2
LANES = 16
DJ = D // LANES
G = 8

TBLK = 1600
NBLK = N_TC // TBLK
BLK0 = N_SC // TBLK
NBLK_TOT = N // TBLK

NEG_INF = float("-inf")

assert R * NW == N_SC and NCHUNK * CHUNK == R
assert NBLK * TBLK == N_TC and BLK0 * TBLK == N_SC


def _tile_body(feat_hbm, ids_hbm, out_hbm,
               ids_v, buf0, buf1, acc_v, cur_v, sem0, sem1):
    c = lax.axis_index("c")
    s = lax.axis_index("s")
    w = s * NC + c
    base = w * R

    pltpu.sync_copy(ids_hbm.at[pl.ds(base, R)], ids_v.at[pl.ds(0, R)])

    def init_body(i, carry):
        acc_v[pl.ds(i * LANES, LANES)] = jnp.full((LANES,), NEG_INF, jnp.float32)
        return carry
    lax.fori_loop(0, (NSEG * D) // LANES, init_body, 0)
    for j in range(DJ):
        cur_v[pl.ds(j * LANES, LANES)] = jnp.full((LANES,), NEG_INF, jnp.float32)

    def flush(cs):
        for j in range(DJ):
            sa = pl.ds(cs * D + j * LANES, LANES)
            sc = pl.ds(j * LANES, LANES)
            acc_v[sa] = jnp.maximum(acc_v[sa], cur_v[sc])
            cur_v[sc] = jnp.full((LANES,), NEG_INF, jnp.float32)

    def process(buf, goff, cs):
        cfirst = ids_v[pl.ds(goff, LANES)][0]
        clast = ids_v[pl.ds(goff + CHUNK - 1, LANES)][0]

        def fast_chunk(cs):
            @pl.when(cfirst != cs)
            def _():
                flush(cs)

            def grp(k, accs):
                out = []
                for j in range(DJ):
                    a = accs[j]
                    for r in range(G):
                        a = jnp.maximum(
                            a, buf[k * G + r, pl.ds(j * LANES, LANES)])
                    out.append(a)
                return tuple(out)

            accs = tuple(cur_v[pl.ds(j * LANES, LANES)] for j in range(DJ))
            accs = lax.fori_loop(0, CHUNK // G, grp, accs)
            for j in range(DJ):
                cur_v[pl.ds(j * LANES, LANES)] = accs[j]
            return cfirst

        def slow_chunk(cs):
            def group_body(k, cs):
                ids16 = ids_v[pl.ds(goff + k * G, LANES)]
                first = ids16[0]
                last = ids16[G - 1]

                def fast(cs):
                    @pl.when(first != cs)
                    def _():
                        flush(cs)
                    for j in range(DJ):
                        sl = pl.ds(j * LANES, LANES)
                        a = cur_v[sl]
                        for r in range(G):
                            a = jnp.maximum(
                                a, buf[k * G + r, pl.ds(j * LANES, LANES)])
                        cur_v[sl] = a
                    return first

                def slow(cs):
                    for r in range(G):
                        soff = ids16[r] * D
                        for j in range(DJ):
                            sl = pl.ds(soff + j * LANES, LANES)
                            acc_v[sl] = jnp.maximum(
                                acc_v[sl],
                                buf[k * G + r, pl.ds(j * LANES, LANES)])
                    return cs

                return lax.cond(first == last, fast, slow, cs)
            return lax.fori_loop(0, CHUNK // G, group_body, cs)

        return lax.cond(cfirst == clast, fast_chunk, slow_chunk, cs)

    pltpu.async_copy(feat_hbm.at[pl.ds(base, CHUNK)], buf0, sem0)

    cs = ids_v[pl.ds(0, LANES)][0]

    def chunk_pair(g, cs):
        pltpu.async_copy(
            feat_hbm.at[pl.ds(base + (2 * g + 1) * CHUNK, CHUNK)], buf1, sem1)
        pltpu.make_async_copy(
            feat_hbm.at[pl.ds(base, CHUNK)], buf0, sem0).wait()
        cs = process(buf0, 2 * g * CHUNK, cs)

        @pl.when(2 * g + 2 < NCHUNK)
        def _():
            pltpu.async_copy(
                feat_hbm.at[pl.ds(base + (2 * g + 2) * CHUNK, CHUNK)],
                buf0, sem0)

        pltpu.make_async_copy(
            feat_hbm.at[pl.ds(base, CHUNK)], buf1, sem1).wait()
        cs = process(buf1, (2 * g + 1) * CHUNK, cs)
        return cs

    cs = lax.fori_loop(0, NPAIR, chunk_pair, cs)

    if NCHUNK % 2 == 1:
        pltpu.make_async_copy(feat_hbm.at[pl.ds(base, CHUNK)], buf0, sem0).wait()
        cs = process(buf0, (NCHUNK - 1) * CHUNK, cs)
    flush(cs)

    pltpu.sync_copy(acc_v, out_hbm.at[w])


@functools.partial(
    pl.kernel,
    out_type=jax.ShapeDtypeStruct((NW, NSEG * D), jnp.float32),
    mesh=plsc.VectorSubcoreMesh(core_axis_name="c", subcore_axis_name="s"),
    scratch_types=[
        pltpu.VMEM((R + LANES,), jnp.int32),
        pltpu.VMEM((CHUNK, D), jnp.float32),
        pltpu.VMEM((CHUNK, D), jnp.float32),
        pltpu.VMEM((NSEG * D,), jnp.float32),
        pltpu.VMEM((D,), jnp.float32),
        pltpu.SemaphoreType.DMA,
        pltpu.SemaphoreType.DMA,
    ],
)
def _segmax_sc(feat_hbm, ids_hbm, out_hbm,
               ids_v, buf0, buf1, acc_v, cur_v, sem0, sem1):
    _tile_body(feat_hbm, ids_hbm, out_hbm,
               ids_v, buf0, buf1, acc_v, cur_v, sem0, sem1)


def _tc_body(ids_ref, feat_ref, out_ref):
    i = pl.program_id(0)

    @pl.when(i == 0)
    def _():
        out_ref[...] = jnp.full((NSEG, D), NEG_INF, jnp.float32)

    ids_row = ids_ref[0]
    lo = jnp.min(ids_row)
    hi = jnp.max(ids_row)

    rows = lax.broadcasted_iota(jnp.int32, (NSEG, D), 0)

    def fold(seg, smax):
        cur = out_ref[...]
        out_ref[...] = jnp.where(rows == seg, jnp.maximum(cur, smax), cur)

    def uniform():
        fold(lo, jnp.max(feat_ref[...], axis=0, keepdims=True))

    def general():
        x = feat_ref[...]
        riota = lax.broadcasted_iota(jnp.int32, (TBLK, D), 0)

        def seg_body(s, carry):
            p0 = jnp.sum(jnp.where(ids_row < s, 1, 0))
            p1 = jnp.sum(jnp.where(ids_row <= s, 1, 0))
            m = (riota >= p0) & (riota < p1)
            fold(s, jnp.max(jnp.where(m, x, NEG_INF), axis=0, keepdims=True))
            return carry
        lax.fori_loop(lo, hi + 1, seg_body, 0)

    lax.cond(lo == hi, uniform, general)


def _segmax_tc(features, ids_row3):
    return pl.pallas_call(
        _tc_body,
        grid=(NBLK,),
        in_specs=[
            pl.BlockSpec((1, 1, TBLK), lambda i: (BLK0 + i, 0, 0)),
            pl.BlockSpec((TBLK, D), lambda i: (BLK0 + i, 0)),
        ],
        out_specs=pl.BlockSpec((NSEG, D), lambda i: (0, 0)),
        out_shape=jax.ShapeDtypeStruct((NSEG, D), jnp.float32),
    )(ids_row3, features)


def kernel(features, segment_ids):
    ids32 = segment_ids.astype(jnp.int32)
    parts = _segmax_sc(features, ids32)
    sc_out = jnp.max(parts.reshape(NW, NSEG, D), axis=0)
    ids_row3 = ids32.reshape(NBLK_TOT, 1, TBLK)
    tc_out = _segmax_tc(features, ids_row3)
    return jnp.maximum(sc_out, tc_out)


# speedup vs baseline: 2.4907x; 1.1217x over previous
import functools

import jax
import jax.numpy as jnp
from jax import lax
from jax.experimental import pallas as pl
from jax.experimental.pallas import tpu as pltpu
from jax.experimental.pallas import tpu_sc as plsc

N = 160000
D = 256
NSEG = 64

N_SC = 89600
N_TC = N - N_SC

NC = 2
NS = 16
NW = NC * NS
R = N_SC // NW
CHUNK = 200
NCHUNK = R // CHUNK
NPAIR = NCHUNK // 2
LANES = 16
DJ = D // LANES
G = 8

TBLK = 1600
NBLK = N_TC // TBLK
BLK0 = N_SC // TBLK
NBLK_TOT = N // TBLK

NEG_INF = float("-inf")

assert R * NW == N_SC and NCHUNK * CHUNK == R
assert NBLK * TBLK == N_TC and BLK0 * TBLK == N_SC


def _tile_body(feat_hbm, ids_hbm, out_hbm,
               ids_v, buf0, buf1, acc_v, cur_v, sem0, sem1):
    c = lax.axis_index("c")
    s = lax.axis_index("s")
    w = s * NC + c
    base = w * R

    pltpu.sync_copy(ids_hbm.at[pl.ds(base, R)], ids_v.at[pl.ds(0, R)])

    def init_body(i, carry):
        acc_v[pl.ds(i * LANES, LANES)] = jnp.full((LANES,), NEG_INF, jnp.float32)
        return carry
    lax.fori_loop(0, (NSEG * D) // LANES, init_body, 0)
    for j in range(DJ):
        cur_v[pl.ds(j * LANES, LANES)] = jnp.full((LANES,), NEG_INF, jnp.float32)

    def flush(cs):
        for j in range(DJ):
            sa = pl.ds(cs * D + j * LANES, LANES)
            sc = pl.ds(j * LANES, LANES)
            acc_v[sa] = jnp.maximum(acc_v[sa], cur_v[sc])
            cur_v[sc] = jnp.full((LANES,), NEG_INF, jnp.float32)

    def process(buf, goff, cs):
        cfirst = ids_v[pl.ds(goff, LANES)][0]
        clast = ids_v[pl.ds(goff + CHUNK - 1, LANES)][0]

        def fast_chunk(cs):
            @pl.when(cfirst != cs)
            def _():
                flush(cs)

            def grp(k, accs):
                out = []
                for j in range(DJ):
                    a = accs[j]
                    for r in range(G):
                        a = jnp.maximum(
                            a, buf[k * G + r, pl.ds(j * LANES, LANES)])
                    out.append(a)
                return tuple(out)

            accs = tuple(cur_v[pl.ds(j * LANES, LANES)] for j in range(DJ))
            accs = lax.fori_loop(0, CHUNK // G, grp, accs)
            for j in range(DJ):
                cur_v[pl.ds(j * LANES, LANES)] = accs[j]
            return cfirst

        def slow_chunk(cs):
            def group_body(k, cs):
                ids16 = ids_v[pl.ds(goff + k * G, LANES)]
                first = ids16[0]
                last = ids16[G - 1]

                def fast(cs):
                    @pl.when(first != cs)
                    def _():
                        flush(cs)
                    for j in range(DJ):
                        sl = pl.ds(j * LANES, LANES)
                        a = cur_v[sl]
                        for r in range(G):
                            a = jnp.maximum(
                                a, buf[k * G + r, pl.ds(j * LANES, LANES)])
                        cur_v[sl] = a
                    return first

                def slow(cs):
                    for r in range(G):
                        soff = ids16[r] * D
                        for j in range(DJ):
                            sl = pl.ds(soff + j * LANES, LANES)
                            acc_v[sl] = jnp.maximum(
                                acc_v[sl],
                                buf[k * G + r, pl.ds(j * LANES, LANES)])
                    return cs

                return lax.cond(first == last, fast, slow, cs)
            return lax.fori_loop(0, CHUNK // G, group_body, cs)

        return lax.cond(cfirst == clast, fast_chunk, slow_chunk, cs)

    pltpu.async_copy(feat_hbm.at[pl.ds(base, CHUNK)], buf0, sem0)

    cs = ids_v[pl.ds(0, LANES)][0]

    def chunk_pair(g, cs):
        pltpu.async_copy(
            feat_hbm.at[pl.ds(base + (2 * g + 1) * CHUNK, CHUNK)], buf1, sem1)
        pltpu.make_async_copy(
            feat_hbm.at[pl.ds(base, CHUNK)], buf0, sem0).wait()
        cs = process(buf0, 2 * g * CHUNK, cs)

        @pl.when(2 * g + 2 < NCHUNK)
        def _():
            pltpu.async_copy(
                feat_hbm.at[pl.ds(base + (2 * g + 2) * CHUNK, CHUNK)],
                buf0, sem0)

        pltpu.make_async_copy(
            feat_hbm.at[pl.ds(base, CHUNK)], buf1, sem1).wait()
        cs = process(buf1, (2 * g + 1) * CHUNK, cs)
        return cs

    cs = lax.fori_loop(0, NPAIR, chunk_pair, cs)

    if NCHUNK % 2 == 1:
        pltpu.make_async_copy(feat_hbm.at[pl.ds(base, CHUNK)], buf0, sem0).wait()
        cs = process(buf0, (NCHUNK - 1) * CHUNK, cs)
    flush(cs)

    pltpu.sync_copy(acc_v, out_hbm.at[w])


@functools.partial(
    pl.kernel,
    out_type=jax.ShapeDtypeStruct((NW, NSEG * D), jnp.float32),
    mesh=plsc.VectorSubcoreMesh(core_axis_name="c", subcore_axis_name="s"),
    scratch_types=[
        pltpu.VMEM((R + LANES,), jnp.int32),
        pltpu.VMEM((CHUNK, D), jnp.float32),
        pltpu.VMEM((CHUNK, D), jnp.float32),
        pltpu.VMEM((NSEG * D,), jnp.float32),
        pltpu.VMEM((D,), jnp.float32),
        pltpu.SemaphoreType.DMA,
        pltpu.SemaphoreType.DMA,
    ],
)
def _segmax_sc(feat_hbm, ids_hbm, out_hbm,
               ids_v, buf0, buf1, acc_v, cur_v, sem0, sem1):
    _tile_body(feat_hbm, ids_hbm, out_hbm,
               ids_v, buf0, buf1, acc_v, cur_v, sem0, sem1)


def _tc_body(ids_ref, feat_ref, out_ref):
    i = pl.program_id(0)

    @pl.when(i == 0)
    def _():
        out_ref[...] = jnp.full((NSEG, D), NEG_INF, jnp.float32)

    ids_row = ids_ref[0]
    lo = jnp.min(ids_row)
    hi = jnp.max(ids_row)

    rows = lax.broadcasted_iota(jnp.int32, (NSEG, D), 0)

    def fold(seg, smax):
        cur = out_ref[...]
        out_ref[...] = jnp.where(rows == seg, jnp.maximum(cur, smax), cur)

    def uniform():
        fold(lo, jnp.max(feat_ref[...], axis=0, keepdims=True))

    def general():
        x = feat_ref[...]
        riota = lax.broadcasted_iota(jnp.int32, (TBLK, D), 0)

        def seg_body(s, carry):
            p0 = jnp.sum(jnp.where(ids_row < s, 1, 0))
            p1 = jnp.sum(jnp.where(ids_row <= s, 1, 0))
            m = (riota >= p0) & (riota < p1)
            fold(s, jnp.max(jnp.where(m, x, NEG_INF), axis=0, keepdims=True))
            return carry
        lax.fori_loop(lo, hi + 1, seg_body, 0)

    lax.cond(lo == hi, uniform, general)


def _segmax_tc(features, ids_row3):
    return pl.pallas_call(
        _tc_body,
        grid=(NBLK,),
        in_specs=[
            pl.BlockSpec((1, 1, TBLK), lambda i: (BLK0 + i, 0, 0)),
            pl.BlockSpec((TBLK, D), lambda i: (BLK0 + i, 0)),
        ],
        out_specs=pl.BlockSpec((NSEG, D), lambda i: (0, 0)),
        out_shape=jax.ShapeDtypeStruct((NSEG, D), jnp.float32),
    )(ids_row3, features)


def kernel(features, segment_ids):
    ids32 = segment_ids.astype(jnp.int32)
    parts = _segmax_sc(features, ids32)
    sc_out = jnp.max(parts.reshape(NW, NSEG, D), axis=0)
    ids_row3 = ids32.reshape(NBLK_TOT, 1, TBLK)
    tc_out = _segmax_tc(features, ids_row3)
    return jnp.maximum(sc_out, tc_out)
